# baseline (speedup 1.0000x reference)
import functools

import jax
import jax.numpy as jnp
from jax import lax
from jax.experimental import pallas as pl
from jax.experimental.pallas import tpu as pltpu
from jax.experimental.pallas import tpu_sc as plsc

N_IMGS = 64
N_OUT = 128
D = 3 * 384 * 384
C = 64
CW = D // C
NW = 32
IPW = N_OUT * C // NW
RB = 8
NB = IPW // RB

_mesh = plsc.VectorSubcoreMesh(core_axis_name="c", subcore_axis_name="s")


@functools.partial(
    pl.kernel,
    mesh=_mesh,
    out_type=jax.ShapeDtypeStruct((N_OUT * C, CW), jnp.float32),
    scratch_types=[
        pltpu.VMEM((IPW,), jnp.int32),
        pltpu.VMEM((IPW,), jnp.int32),
        pltpu.VMEM((IPW,), jnp.int32),
        pltpu.VMEM((RB, CW), jnp.float32),
        pltpu.SemaphoreType.DMA,
        pltpu.SemaphoreType.DMA,
    ],
)
def _sc_gather(idx_hbm, table_hbm, out_hbm, ilist_v, rowv_v, src_v, buf,
               isem, gsem):
    wid = lax.axis_index("s") * 2 + lax.axis_index("c")
    base = wid * IPW

    for j in range(IPW // 16):
        ilist_v[pl.ds(j * 16, 16)] = jnp.full(
            (16,), (base + j * 16) // C, jnp.int32)
    pltpu.async_copy(idx_hbm.at[ilist_v], rowv_v, isem).wait()
    lane = lax.broadcasted_iota(jnp.int32, (16,), 0)
    for j in range(IPW // 16):
        sl = pl.ds(j * 16, 16)
        src_v[sl] = rowv_v[sl] * C + ((j * 16) % C + lane)

    def step(k, carry):
        off = pl.multiple_of(k * RB, RB)
        pltpu.async_copy(
            table_hbm.at[src_v.at[pl.ds(off, RB)]], buf, gsem
        ).wait()
        pltpu.sync_copy(buf, out_hbm.at[pl.ds(base + off, RB)])
        return carry

    lax.fori_loop(0, NB, step, 0)


def kernel(idx, imgs):
    idx = idx.astype(jnp.int32)
    table = imgs.reshape(N_IMGS * C, CW)
    out = _sc_gather(idx, table)
    return out.reshape(N_OUT, 3, 384, 384)
